# Initial kernel scaffold; baseline (speedup 1.0000x reference)
import jax
import jax.numpy as jnp
from jax.experimental import pallas as pl


def kernel(query, key, value, g, beta, last_recurrent_state):
    raise NotImplementedError("write your pallas kernel here")


# speedup vs baseline: 5.9601x; 5.9601x over previous
import jax
import jax.numpy as jnp
from jax import lax
from jax.experimental import pallas as pl
from jax.experimental.pallas import tpu as pltpu

_C = 64

_HP = jax.lax.Precision.HIGHEST


def _gdn_kernel(q_ref, k_ref, v_ref, g_ref, b_ref, s0_ref, o_ref, fs_ref):
    c = pl.program_id(1)
    f32 = jnp.float32
    C = _C

    @pl.when(c == 0)
    def _init():
        fs_ref[...] = s0_ref[...]

    q = q_ref[0]
    k = k_ref[0]
    v = v_ref[0]
    g = g_ref[0, 0]
    beta = b_ref[0, 0]
    state = fs_ref[0]

    tt = lax.broadcasted_iota(jnp.int32, (C, C), 0)
    ii = lax.broadcasted_iota(jnp.int32, (C, C), 1)
    cum_mask = (tt <= ii).astype(f32)
    G = jnp.dot(g, cum_mask, preferred_element_type=f32,
                precision=jax.lax.Precision.HIGHEST)

    Gc = G.reshape(C, 1)
    bc = beta.reshape(C, 1)
    expGc = jnp.exp(Gc)

    row = lax.broadcasted_iota(jnp.int32, (C, C), 0)
    col = lax.broadcasted_iota(jnp.int32, (C, C), 1)
    Gdiff = Gc - G
    neg = f32(-1e30)
    d_incl = jnp.exp(jnp.where(row >= col, Gdiff, neg))
    d_strict = jnp.exp(jnp.where(row > col, Gdiff, neg))

    kkT = lax.dot_general(k, k, (((1,), (1,)), ((), ())),
                          preferred_element_type=f32, precision=_HP)
    A = bc * d_strict * kkT

    eye = (row == col).astype(f32)
    T = eye - A
    P = A
    for _ in range(5):
        P = jnp.dot(P, P, preferred_element_type=f32, precision=_HP)
        T = T + jnp.dot(T, P, preferred_element_type=f32, precision=_HP)

    ks = k * expGc
    R = bc * (v - jnp.dot(ks, state, preferred_element_type=f32, precision=_HP))
    U = jnp.dot(T, R, preferred_element_type=f32, precision=_HP)

    qs = q * expGc
    qkT = lax.dot_general(q, k, (((1,), (1,)), ((), ())),
                          preferred_element_type=f32, precision=_HP)
    o = (jnp.dot(qs, state, preferred_element_type=f32, precision=_HP)
         + jnp.dot(d_incl * qkT, U, preferred_element_type=f32, precision=_HP))
    o_ref[0] = o

    Gl = G[:, C - 1:C]
    kd = k * jnp.exp(Gl - Gc)
    fs_ref[0] = jnp.exp(Gl) * state + lax.dot_general(
        kd, U, (((0,), (0,)), ((), ())),
        preferred_element_type=f32, precision=_HP)


def kernel(query, key, value, g, beta, last_recurrent_state):
    B, H, S, D = query.shape
    BH = B * H
    C = _C
    NC = S // C

    q = query.reshape(BH, S, D)
    k = key.reshape(BH, S, D)
    v = value.reshape(BH, S, D)
    g4 = g.reshape(BH, NC, 1, C)
    b4 = beta.reshape(BH, NC, 1, C)
    s0 = last_recurrent_state.reshape(BH, D, D)

    qkv_spec = pl.BlockSpec((1, C, D), lambda h, c: (h, c, 0))
    gb_spec = pl.BlockSpec((1, 1, 1, C), lambda h, c: (h, c, 0, 0))
    st_spec = pl.BlockSpec((1, D, D), lambda h, c: (h, 0, 0))

    o, fs = pl.pallas_call(
        _gdn_kernel,
        grid=(BH, NC),
        in_specs=[qkv_spec, qkv_spec, qkv_spec, gb_spec, gb_spec, st_spec],
        out_specs=[pl.BlockSpec((1, C, D), lambda h, c: (h, c, 0)), st_spec],
        out_shape=[
            jax.ShapeDtypeStruct((BH, S, D), jnp.float32),
            jax.ShapeDtypeStruct((BH, D, D), jnp.float32),
        ],
        compiler_params=pltpu.CompilerParams(
            dimension_semantics=("parallel", "arbitrary"),
        ),
        name="gdn_chunked",
    )(q, k, v, g4, b4, s0)

    return jnp.concatenate([o.reshape(-1), fs.reshape(-1)], axis=0)


# speedup vs baseline: 8.7853x; 1.4740x over previous
import jax
import jax.numpy as jnp
from jax import lax
from jax.experimental import pallas as pl
from jax.experimental.pallas import tpu as pltpu

_C = 64

_HP = jax.lax.Precision.DEFAULT


def _gdn_kernel(q_ref, k_ref, v_ref, g_ref, b_ref, s0_ref, o_ref, fs_ref):
    c = pl.program_id(1)
    f32 = jnp.float32
    C = _C

    @pl.when(c == 0)
    def _init():
        fs_ref[...] = s0_ref[...]

    q = q_ref[0]
    k = k_ref[0]
    v = v_ref[0]
    g = g_ref[0, 0]
    beta = b_ref[0, 0]
    state = fs_ref[0]

    tt = lax.broadcasted_iota(jnp.int32, (C, C), 0)
    ii = lax.broadcasted_iota(jnp.int32, (C, C), 1)
    cum_mask = (tt <= ii).astype(f32)
    G = jnp.dot(g, cum_mask, preferred_element_type=f32,
                precision=jax.lax.Precision.HIGHEST)

    Gc = G.reshape(C, 1)
    bc = beta.reshape(C, 1)
    expGc = jnp.exp(Gc)

    row = lax.broadcasted_iota(jnp.int32, (C, C), 0)
    col = lax.broadcasted_iota(jnp.int32, (C, C), 1)
    Gdiff = Gc - G
    neg = f32(-1e30)
    d_incl = jnp.exp(jnp.where(row >= col, Gdiff, neg))
    d_strict = jnp.exp(jnp.where(row > col, Gdiff, neg))

    kkT = lax.dot_general(k, k, (((1,), (1,)), ((), ())),
                          preferred_element_type=f32, precision=_HP)
    A = bc * d_strict * kkT

    eye = (row == col).astype(f32)
    T = eye - A
    P = A
    for _ in range(5):
        P = jnp.dot(P, P, preferred_element_type=f32, precision=_HP)
        T = T + jnp.dot(T, P, preferred_element_type=f32, precision=_HP)

    ks = k * expGc
    R = bc * (v - jnp.dot(ks, state, preferred_element_type=f32, precision=_HP))
    U = jnp.dot(T, R, preferred_element_type=f32, precision=_HP)

    qs = q * expGc
    qkT = lax.dot_general(q, k, (((1,), (1,)), ((), ())),
                          preferred_element_type=f32, precision=_HP)
    o = (jnp.dot(qs, state, preferred_element_type=f32, precision=_HP)
         + jnp.dot(d_incl * qkT, U, preferred_element_type=f32, precision=_HP))
    o_ref[0] = o

    Gl = G[:, C - 1:C]
    kd = k * jnp.exp(Gl - Gc)
    fs_ref[0] = jnp.exp(Gl) * state + lax.dot_general(
        kd, U, (((0,), (0,)), ((), ())),
        preferred_element_type=f32, precision=_HP)


def kernel(query, key, value, g, beta, last_recurrent_state):
    B, H, S, D = query.shape
    BH = B * H
    C = _C
    NC = S // C

    q = query.reshape(BH, S, D)
    k = key.reshape(BH, S, D)
    v = value.reshape(BH, S, D)
    g4 = g.reshape(BH, NC, 1, C)
    b4 = beta.reshape(BH, NC, 1, C)
    s0 = last_recurrent_state.reshape(BH, D, D)

    qkv_spec = pl.BlockSpec((1, C, D), lambda h, c: (h, c, 0))
    gb_spec = pl.BlockSpec((1, 1, 1, C), lambda h, c: (h, c, 0, 0))
    st_spec = pl.BlockSpec((1, D, D), lambda h, c: (h, 0, 0))

    o, fs = pl.pallas_call(
        _gdn_kernel,
        grid=(BH, NC),
        in_specs=[qkv_spec, qkv_spec, qkv_spec, gb_spec, gb_spec, st_spec],
        out_specs=[pl.BlockSpec((1, C, D), lambda h, c: (h, c, 0)), st_spec],
        out_shape=[
            jax.ShapeDtypeStruct((BH, S, D), jnp.float32),
            jax.ShapeDtypeStruct((BH, D, D), jnp.float32),
        ],
        compiler_params=pltpu.CompilerParams(
            dimension_semantics=("parallel", "arbitrary"),
        ),
        name="gdn_chunked",
    )(q, k, v, g4, b4, s0)

    return jnp.concatenate([o.reshape(-1), fs.reshape(-1)], axis=0)


# speedup vs baseline: 9.4869x; 1.0799x over previous
import jax
import jax.numpy as jnp
from jax import lax
from jax.experimental import pallas as pl
from jax.experimental.pallas import tpu as pltpu

_C = 64
_HB = 4

_HP = jax.lax.Precision.DEFAULT


def _gdn_kernel(q_ref, k_ref, v_ref, g_ref, b_ref, s0_ref, o_ref, fs_ref):
    c = pl.program_id(1)
    f32 = jnp.float32
    C = _C

    @pl.when(c == 0)
    def _init():
        fs_ref[...] = s0_ref[...]

    tt = lax.broadcasted_iota(jnp.int32, (C, C), 0)
    ii = lax.broadcasted_iota(jnp.int32, (C, C), 1)
    cum_mask = (tt <= ii).astype(f32)
    eye = (tt == ii).astype(f32)
    neg = f32(-1e30)

    for h in range(_HB):
        q = q_ref[h]
        k = k_ref[h]
        v = v_ref[h]
        g = g_ref[h, 0]
        beta = b_ref[h, 0]
        state = fs_ref[h]

        G = jnp.dot(g, cum_mask, preferred_element_type=f32,
                    precision=jax.lax.Precision.HIGHEST)

        Gc = G.reshape(C, 1)
        bc = beta.reshape(C, 1)
        expGc = jnp.exp(Gc)

        Gdiff = Gc - G
        d_incl = jnp.exp(jnp.where(tt >= ii, Gdiff, neg))
        d_strict = jnp.exp(jnp.where(tt > ii, Gdiff, neg))

        kkT = lax.dot_general(k, k, (((1,), (1,)), ((), ())),
                              preferred_element_type=f32, precision=_HP)
        A = bc * d_strict * kkT

        T = eye - A
        P = A
        for _ in range(5):
            P = jnp.dot(P, P, preferred_element_type=f32, precision=_HP)
            T = T + jnp.dot(T, P, preferred_element_type=f32, precision=_HP)

        ks = k * expGc
        R = bc * (v - jnp.dot(ks, state, preferred_element_type=f32,
                              precision=_HP))
        U = jnp.dot(T, R, preferred_element_type=f32, precision=_HP)

        qs = q * expGc
        qkT = lax.dot_general(q, k, (((1,), (1,)), ((), ())),
                              preferred_element_type=f32, precision=_HP)
        o = (jnp.dot(qs, state, preferred_element_type=f32, precision=_HP)
             + jnp.dot(d_incl * qkT, U, preferred_element_type=f32,
                       precision=_HP))
        o_ref[h] = o

        Gl = G[:, C - 1:C]
        kd = k * jnp.exp(Gl - Gc)
        fs_ref[h] = jnp.exp(Gl) * state + lax.dot_general(
            kd, U, (((0,), (0,)), ((), ())),
            preferred_element_type=f32, precision=_HP)


def kernel(query, key, value, g, beta, last_recurrent_state):
    B, H, S, D = query.shape
    BH = B * H
    C = _C
    HB = _HB
    NC = S // C
    NH = BH // HB

    q = query.reshape(BH, S, D)
    k = key.reshape(BH, S, D)
    v = value.reshape(BH, S, D)
    g4 = g.reshape(BH, NC, 1, C)
    b4 = beta.reshape(BH, NC, 1, C)
    s0 = last_recurrent_state.reshape(BH, D, D)

    qkv_spec = pl.BlockSpec((HB, C, D), lambda h, c: (h, c, 0))
    gb_spec = pl.BlockSpec((HB, 1, 1, C), lambda h, c: (h, c, 0, 0))
    st_spec = pl.BlockSpec((HB, D, D), lambda h, c: (h, 0, 0))

    o, fs = pl.pallas_call(
        _gdn_kernel,
        grid=(NH, NC),
        in_specs=[qkv_spec, qkv_spec, qkv_spec, gb_spec, gb_spec, st_spec],
        out_specs=[pl.BlockSpec((HB, C, D), lambda h, c: (h, c, 0)), st_spec],
        out_shape=[
            jax.ShapeDtypeStruct((BH, S, D), jnp.float32),
            jax.ShapeDtypeStruct((BH, D, D), jnp.float32),
        ],
        compiler_params=pltpu.CompilerParams(
            dimension_semantics=("parallel", "arbitrary"),
        ),
        name="gdn_chunked",
    )(q, k, v, g4, b4, s0)

    return jnp.concatenate([o.reshape(-1), fs.reshape(-1)], axis=0)


# speedup vs baseline: 29.9692x; 3.1590x over previous
import jax
import jax.numpy as jnp
from jax import lax
from jax.experimental import pallas as pl
from jax.experimental.pallas import tpu as pltpu

_C = 64
_HB = 4

_HP = jax.lax.Precision.DEFAULT
_F32 = jnp.float32


def _dot(a, b, prec=_HP):
    return jnp.dot(a, b, preferred_element_type=_F32, precision=prec)


def _dot_t(a, b, dims, prec=_HP):
    return lax.dot_general(a, b, (dims, ((), ())),
                           preferred_element_type=_F32, precision=prec)


def _gdn_kernel(q_ref, k_ref, v_ref, g_ref, b_ref, s0_ref, o_ref, fs_ref):
    c = pl.program_id(1)
    C = _C
    HB = _HB

    @pl.when(c == 0)
    def _init():
        fs_ref[...] = s0_ref[...]

    tt = lax.broadcasted_iota(jnp.int32, (C, C), 0)
    ii = lax.broadcasted_iota(jnp.int32, (C, C), 1)
    cum_mask = (tt <= ii).astype(_F32)
    eye = (tt == ii).astype(_F32)
    neg = _F32(-1e30)

    ks, qs, vs, states = [], [], [], []
    Gcs, Gls, bcs = [], [], []
    d_strict, d_inclT = [], []
    for h in range(HB):
        k = k_ref[h]
        q = q_ref[h]
        ks.append(k)
        qs.append(q)
        vs.append(v_ref[h])
        states.append(fs_ref[h])
        g = g_ref[h, 0]
        beta = b_ref[h, 0]
        G = _dot(g, cum_mask, prec=jax.lax.Precision.HIGHEST)
        Gc = G.reshape(C, 1)
        Gcs.append(Gc)
        Gls.append(G[:, C - 1:C])
        bcs.append(beta.reshape(C, 1))
        Gdiff = Gc - G
        d_strict.append(jnp.exp(jnp.where(tt > ii, Gdiff, neg)))
        d_inclT.append(jnp.exp(jnp.where(ii >= tt, G - Gc, neg)))

    KK = [_dot_t(ks[h], jnp.concatenate([ks[h], qs[h]], axis=0), ((1,), (1,)))
          for h in range(HB)]

    SP = [_dot(jnp.concatenate([ks[h] * jnp.exp(Gcs[h]),
                                qs[h] * jnp.exp(Gcs[h])], axis=0), states[h])
          for h in range(HB)]

    Ts, Ps = [], []
    for h in range(HB):
        A = bcs[h] * d_strict[h] * KK[h][:, :C]
        Ts.append(eye - A)
        Ps.append(A)
    for _ in range(5):
        for h in range(HB):
            Y = _dot(jnp.concatenate([Ts[h], Ps[h]], axis=0), Ps[h])
            Ts[h] = Ts[h] + Y[:C]
            Ps[h] = Y[C:]

    Us = []
    for h in range(HB):
        R = bcs[h] * (vs[h] - SP[h][:C])
        Us.append(_dot(Ts[h], R))

    for h in range(HB):
        attnT = d_inclT[h] * KK[h][:, C:]
        kd = ks[h] * jnp.exp(Gls[h] - Gcs[h])
        Z = _dot_t(jnp.concatenate([attnT, kd], axis=1), Us[h],
                   ((0,), (0,)))
        o_ref[h] = SP[h][C:] + Z[:C]
        fs_ref[h] = jnp.exp(Gls[h]) * states[h] + Z[C:]


def kernel(query, key, value, g, beta, last_recurrent_state):
    B, H, S, D = query.shape
    BH = B * H
    C = _C
    HB = _HB
    NC = S // C
    NH = BH // HB

    q = query.reshape(BH, S, D)
    k = key.reshape(BH, S, D)
    v = value.reshape(BH, S, D)
    g4 = g.reshape(BH, NC, 1, C)
    b4 = beta.reshape(BH, NC, 1, C)
    s0 = last_recurrent_state.reshape(BH, D, D)

    qkv_spec = pl.BlockSpec((HB, C, D), lambda h, c: (h, c, 0))
    gb_spec = pl.BlockSpec((HB, 1, 1, C), lambda h, c: (h, c, 0, 0))
    st_spec = pl.BlockSpec((HB, D, D), lambda h, c: (h, 0, 0))

    o, fs = pl.pallas_call(
        _gdn_kernel,
        grid=(NH, NC),
        in_specs=[qkv_spec, qkv_spec, qkv_spec, gb_spec, gb_spec, st_spec],
        out_specs=[pl.BlockSpec((HB, C, D), lambda h, c: (h, c, 0)), st_spec],
        out_shape=[
            jax.ShapeDtypeStruct((BH, S, D), jnp.float32),
            jax.ShapeDtypeStruct((BH, D, D), jnp.float32),
        ],
        compiler_params=pltpu.CompilerParams(
            dimension_semantics=("parallel", "arbitrary"),
        ),
        name="gdn_chunked",
    )(q, k, v, g4, b4, s0)

    return jnp.concatenate([o.reshape(-1), fs.reshape(-1)], axis=0)


# speedup vs baseline: 48.0124x; 1.6021x over previous
import jax
import jax.numpy as jnp
from jax import lax
from jax.experimental import pallas as pl
from jax.experimental.pallas import tpu as pltpu

_C = 64
_HB = 8

_HP = jax.lax.Precision.DEFAULT
_F32 = jnp.float32


def _dot(a, b, prec=_HP):
    return jnp.dot(a, b, preferred_element_type=_F32, precision=prec)


def _dot_t(a, b, dims, prec=_HP):
    return lax.dot_general(a, b, (dims, ((), ())),
                           preferred_element_type=_F32, precision=prec)


def _gdn_kernel(q_ref, k_ref, v_ref, g_ref, b_ref, s0_ref, o_ref, fs_ref):
    c = pl.program_id(1)
    C = _C
    HB = _HB

    @pl.when(c == 0)
    def _init():
        fs_ref[...] = s0_ref[...]

    tt = lax.broadcasted_iota(jnp.int32, (C, C), 0)
    ii = lax.broadcasted_iota(jnp.int32, (C, C), 1)
    cum_mask = (tt <= ii).astype(_F32)
    eye = (tt == ii).astype(_F32)
    neg = _F32(-1e30)

    ks, qs, vs, states = [], [], [], []
    Gcs, Gls, bcs = [], [], []
    d_strict, d_inclT = [], []
    for h in range(HB):
        k = k_ref[h]
        q = q_ref[h]
        ks.append(k)
        qs.append(q)
        vs.append(v_ref[h])
        states.append(fs_ref[h])
        g = g_ref[h, 0]
        beta = b_ref[h, 0]
        G = _dot(g, cum_mask, prec=jax.lax.Precision.HIGHEST)
        Gc = G.reshape(C, 1)
        Gcs.append(Gc)
        Gls.append(G[:, C - 1:C])
        bcs.append(beta.reshape(C, 1))
        Gdiff = Gc - G
        d_strict.append(jnp.exp(jnp.where(tt > ii, Gdiff, neg)))
        d_inclT.append(jnp.exp(jnp.where(ii >= tt, G - Gc, neg)))

    KK = [_dot_t(ks[h], jnp.concatenate([ks[h], qs[h]], axis=0), ((1,), (1,)))
          for h in range(HB)]

    SP = [_dot(jnp.concatenate([ks[h] * jnp.exp(Gcs[h]),
                                qs[h] * jnp.exp(Gcs[h])], axis=0), states[h])
          for h in range(HB)]

    Ts, Ps = [], []
    for h in range(HB):
        A = bcs[h] * d_strict[h] * KK[h][:, :C]
        Ts.append(eye - A)
        Ps.append(A)
    for _ in range(5):
        for h in range(HB):
            Y = _dot(jnp.concatenate([Ts[h], Ps[h]], axis=0), Ps[h])
            Ts[h] = Ts[h] + Y[:C]
            Ps[h] = Y[C:]

    Us = []
    for h in range(HB):
        R = bcs[h] * (vs[h] - SP[h][:C])
        Us.append(_dot(Ts[h], R))

    for h in range(HB):
        attnT = d_inclT[h] * KK[h][:, C:]
        kd = ks[h] * jnp.exp(Gls[h] - Gcs[h])
        Z = _dot_t(jnp.concatenate([attnT, kd], axis=1), Us[h],
                   ((0,), (0,)))
        o_ref[h] = SP[h][C:] + Z[:C]
        fs_ref[h] = jnp.exp(Gls[h]) * states[h] + Z[C:]


def kernel(query, key, value, g, beta, last_recurrent_state):
    B, H, S, D = query.shape
    BH = B * H
    C = _C
    HB = _HB
    NC = S // C
    NH = BH // HB

    q = query.reshape(BH, S, D)
    k = key.reshape(BH, S, D)
    v = value.reshape(BH, S, D)
    g4 = g.reshape(BH, NC, 1, C)
    b4 = beta.reshape(BH, NC, 1, C)
    s0 = last_recurrent_state.reshape(BH, D, D)

    qkv_spec = pl.BlockSpec((HB, C, D), lambda h, c: (h, c, 0))
    gb_spec = pl.BlockSpec((HB, 1, 1, C), lambda h, c: (h, c, 0, 0))
    st_spec = pl.BlockSpec((HB, D, D), lambda h, c: (h, 0, 0))

    o, fs = pl.pallas_call(
        _gdn_kernel,
        grid=(NH, NC),
        in_specs=[qkv_spec, qkv_spec, qkv_spec, gb_spec, gb_spec, st_spec],
        out_specs=[pl.BlockSpec((HB, C, D), lambda h, c: (h, c, 0)), st_spec],
        out_shape=[
            jax.ShapeDtypeStruct((BH, S, D), jnp.float32),
            jax.ShapeDtypeStruct((BH, D, D), jnp.float32),
        ],
        compiler_params=pltpu.CompilerParams(
            dimension_semantics=("parallel", "arbitrary"),
        ),
        name="gdn_chunked",
    )(q, k, v, g4, b4, s0)

    return jnp.concatenate([o.reshape(-1), fs.reshape(-1)], axis=0)


# speedup vs baseline: 58.1858x; 1.2119x over previous
import jax
import jax.numpy as jnp
from jax import lax
from jax.experimental import pallas as pl
from jax.experimental.pallas import tpu as pltpu

_C = 64
_HB = 16

_HP = jax.lax.Precision.DEFAULT
_F32 = jnp.float32


def _dot(a, b, prec=_HP):
    return jnp.dot(a, b, preferred_element_type=_F32, precision=prec)


def _dot_t(a, b, dims, prec=_HP):
    return lax.dot_general(a, b, (dims, ((), ())),
                           preferred_element_type=_F32, precision=prec)


def _gdn_kernel(q_ref, k_ref, v_ref, g_ref, b_ref, s0_ref, o_ref, fs_ref):
    c = pl.program_id(1)
    C = _C
    HB = _HB

    @pl.when(c == 0)
    def _init():
        fs_ref[...] = s0_ref[...]

    tt = lax.broadcasted_iota(jnp.int32, (C, C), 0)
    ii = lax.broadcasted_iota(jnp.int32, (C, C), 1)
    cum_mask = (tt <= ii).astype(_F32)
    eye = (tt == ii).astype(_F32)
    neg = _F32(-1e30)

    ks, qs, vs, states = [], [], [], []
    Gcs, Gls, bcs = [], [], []
    d_strict, d_inclT = [], []
    for h in range(HB):
        k = k_ref[h]
        q = q_ref[h]
        ks.append(k)
        qs.append(q)
        vs.append(v_ref[h])
        states.append(fs_ref[h])
        g = g_ref[h, 0]
        beta = b_ref[h, 0]
        G = _dot(g, cum_mask, prec=jax.lax.Precision.HIGHEST)
        Gc = G.reshape(C, 1)
        Gcs.append(Gc)
        Gls.append(G[:, C - 1:C])
        bcs.append(beta.reshape(C, 1))
        Gdiff = Gc - G
        d_strict.append(jnp.exp(jnp.where(tt > ii, Gdiff, neg)))
        d_inclT.append(jnp.exp(jnp.where(ii >= tt, G - Gc, neg)))

    KK = [_dot_t(ks[h], jnp.concatenate([ks[h], qs[h]], axis=0), ((1,), (1,)))
          for h in range(HB)]

    SP = [_dot(jnp.concatenate([ks[h] * jnp.exp(Gcs[h]),
                                qs[h] * jnp.exp(Gcs[h])], axis=0), states[h])
          for h in range(HB)]

    Ts, Ps = [], []
    for h in range(HB):
        A = bcs[h] * d_strict[h] * KK[h][:, :C]
        Ts.append(eye - A)
        Ps.append(A)
    for _ in range(5):
        for h in range(HB):
            Y = _dot(jnp.concatenate([Ts[h], Ps[h]], axis=0), Ps[h])
            Ts[h] = Ts[h] + Y[:C]
            Ps[h] = Y[C:]

    Us = []
    for h in range(HB):
        R = bcs[h] * (vs[h] - SP[h][:C])
        Us.append(_dot(Ts[h], R))

    for h in range(HB):
        attnT = d_inclT[h] * KK[h][:, C:]
        kd = ks[h] * jnp.exp(Gls[h] - Gcs[h])
        Z = _dot_t(jnp.concatenate([attnT, kd], axis=1), Us[h],
                   ((0,), (0,)))
        o_ref[h] = SP[h][C:] + Z[:C]
        fs_ref[h] = jnp.exp(Gls[h]) * states[h] + Z[C:]


def kernel(query, key, value, g, beta, last_recurrent_state):
    B, H, S, D = query.shape
    BH = B * H
    C = _C
    HB = _HB
    NC = S // C
    NH = BH // HB

    q = query.reshape(BH, S, D)
    k = key.reshape(BH, S, D)
    v = value.reshape(BH, S, D)
    g4 = g.reshape(BH, NC, 1, C)
    b4 = beta.reshape(BH, NC, 1, C)
    s0 = last_recurrent_state.reshape(BH, D, D)

    qkv_spec = pl.BlockSpec((HB, C, D), lambda h, c: (h, c, 0))
    gb_spec = pl.BlockSpec((HB, 1, 1, C), lambda h, c: (h, c, 0, 0))
    st_spec = pl.BlockSpec((HB, D, D), lambda h, c: (h, 0, 0))

    o, fs = pl.pallas_call(
        _gdn_kernel,
        grid=(NH, NC),
        in_specs=[qkv_spec, qkv_spec, qkv_spec, gb_spec, gb_spec, st_spec],
        out_specs=[pl.BlockSpec((HB, C, D), lambda h, c: (h, c, 0)), st_spec],
        out_shape=[
            jax.ShapeDtypeStruct((BH, S, D), jnp.float32),
            jax.ShapeDtypeStruct((BH, D, D), jnp.float32),
        ],
        compiler_params=pltpu.CompilerParams(
            dimension_semantics=("parallel", "arbitrary"),
        ),
        name="gdn_chunked",
    )(q, k, v, g4, b4, s0)

    return jnp.concatenate([o.reshape(-1), fs.reshape(-1)], axis=0)


# speedup vs baseline: 61.7629x; 1.0615x over previous
import jax
import jax.numpy as jnp
from jax import lax
from jax.experimental import pallas as pl
from jax.experimental.pallas import tpu as pltpu

_C = 64
_HB = 32

_HP = jax.lax.Precision.DEFAULT
_F32 = jnp.float32


def _dot(a, b, prec=_HP):
    return jnp.dot(a, b, preferred_element_type=_F32, precision=prec)


def _dot_t(a, b, dims, prec=_HP):
    return lax.dot_general(a, b, (dims, ((), ())),
                           preferred_element_type=_F32, precision=prec)


def _gdn_kernel(q_ref, k_ref, v_ref, g_ref, b_ref, s0_ref, o_ref, fs_ref):
    c = pl.program_id(1)
    C = _C
    HB = _HB

    @pl.when(c == 0)
    def _init():
        fs_ref[...] = s0_ref[...]

    tt = lax.broadcasted_iota(jnp.int32, (C, C), 0)
    ii = lax.broadcasted_iota(jnp.int32, (C, C), 1)
    cum_mask = (tt <= ii).astype(_F32)
    eye = (tt == ii).astype(_F32)
    neg = _F32(-1e30)

    ks, qs, vs, states = [], [], [], []
    Gcs, Gls, bcs = [], [], []
    d_strict, d_inclT = [], []
    for h in range(HB):
        k = k_ref[h]
        q = q_ref[h]
        ks.append(k)
        qs.append(q)
        vs.append(v_ref[h])
        states.append(fs_ref[h])
        g = g_ref[h, 0]
        beta = b_ref[h, 0]
        G = _dot(g, cum_mask, prec=jax.lax.Precision.HIGHEST)
        Gc = G.reshape(C, 1)
        Gcs.append(Gc)
        Gls.append(G[:, C - 1:C])
        bcs.append(beta.reshape(C, 1))
        Gdiff = Gc - G
        d_strict.append(jnp.exp(jnp.where(tt > ii, Gdiff, neg)))
        d_inclT.append(jnp.exp(jnp.where(ii >= tt, G - Gc, neg)))

    KK = [_dot_t(ks[h], jnp.concatenate([ks[h], qs[h]], axis=0), ((1,), (1,)))
          for h in range(HB)]

    SP = [_dot(jnp.concatenate([ks[h] * jnp.exp(Gcs[h]),
                                qs[h] * jnp.exp(Gcs[h])], axis=0), states[h])
          for h in range(HB)]

    Ts, Ps = [], []
    for h in range(HB):
        A = bcs[h] * d_strict[h] * KK[h][:, :C]
        Ts.append(eye - A)
        Ps.append(A)
    for _ in range(5):
        for h in range(HB):
            Y = _dot(jnp.concatenate([Ts[h], Ps[h]], axis=0), Ps[h])
            Ts[h] = Ts[h] + Y[:C]
            Ps[h] = Y[C:]

    Us = []
    for h in range(HB):
        R = bcs[h] * (vs[h] - SP[h][:C])
        Us.append(_dot(Ts[h], R))

    for h in range(HB):
        attnT = d_inclT[h] * KK[h][:, C:]
        kd = ks[h] * jnp.exp(Gls[h] - Gcs[h])
        Z = _dot_t(jnp.concatenate([attnT, kd], axis=1), Us[h],
                   ((0,), (0,)))
        o_ref[h] = SP[h][C:] + Z[:C]
        fs_ref[h] = jnp.exp(Gls[h]) * states[h] + Z[C:]


def kernel(query, key, value, g, beta, last_recurrent_state):
    B, H, S, D = query.shape
    BH = B * H
    C = _C
    HB = _HB
    NC = S // C
    NH = BH // HB

    q = query.reshape(BH, S, D)
    k = key.reshape(BH, S, D)
    v = value.reshape(BH, S, D)
    g4 = g.reshape(BH, NC, 1, C)
    b4 = beta.reshape(BH, NC, 1, C)
    s0 = last_recurrent_state.reshape(BH, D, D)

    qkv_spec = pl.BlockSpec((HB, C, D), lambda h, c: (h, c, 0))
    gb_spec = pl.BlockSpec((HB, 1, 1, C), lambda h, c: (h, c, 0, 0))
    st_spec = pl.BlockSpec((HB, D, D), lambda h, c: (h, 0, 0))

    o, fs = pl.pallas_call(
        _gdn_kernel,
        grid=(NH, NC),
        in_specs=[qkv_spec, qkv_spec, qkv_spec, gb_spec, gb_spec, st_spec],
        out_specs=[pl.BlockSpec((HB, C, D), lambda h, c: (h, c, 0)), st_spec],
        out_shape=[
            jax.ShapeDtypeStruct((BH, S, D), jnp.float32),
            jax.ShapeDtypeStruct((BH, D, D), jnp.float32),
        ],
        compiler_params=pltpu.CompilerParams(
            dimension_semantics=("parallel", "arbitrary"),
        ),
        name="gdn_chunked",
    )(q, k, v, g4, b4, s0)

    return jnp.concatenate([o.reshape(-1), fs.reshape(-1)], axis=0)
